# baseline (speedup 1.0000x reference)
import functools

import jax
import jax.numpy as jnp
from jax import lax
from jax.experimental import pallas as pl
from jax.experimental.pallas import tpu as pltpu
from jax.experimental.pallas import tpu_sc as plsc

NUM_EMB = 100000
SEM_DIM = 8
D = 64
PAD_IDX = SEM_DIM * NUM_EMB
B, L = 4096, 200
N = B * L

NW = 32
BW = B // NW
LT = L // 8
LANES = 16
SPREAD_MASK = 524288 - 1


def _sc_gather(sem_hbm, tt_hbm, msk_hbm, table_hbm, out_hbm,
               sem_s, tt_s, msk_s, idx_v, stag_v, oblk_v,
               g_sem0, g_sem1, wb_sem0, wb_sem1):
    wid = lax.axis_index("s") * 2 + lax.axis_index("c")
    g_sems = (g_sem0, g_sem1)
    wb_sems = (wb_sem0, wb_sem1)
    iota = lax.iota(jnp.int32, LANES)

    def compute_idx(lt, lp, pb):
        l = lt * 8 + lp
        for g in range(BW // LANES):
            sl = pl.ds(g * LANES, LANES)
            ids = tt_s[lp, sl] * NUM_EMB + sem_s[lp, sl]
            pos = (l * B + wid * BW + g * LANES + iota) & SPREAD_MASK
            idx_v[pb, sl] = jnp.where(msk_s[lp, sl] != 0, ids, pos)

    def issue_gather(pb):
        pltpu.async_copy(table_hbm.at[idx_v.at[pb]], stag_v.at[pb],
                         g_sems[pb])

    def wait_gather(pb):
        pltpu.make_async_copy(table_hbm.at[idx_v.at[pb]], stag_v.at[pb],
                              g_sems[pb]).wait()

    def issue_wb(l, pb):
        pltpu.async_copy(oblk_v.at[pb], out_hbm.at[l, :, wid], wb_sems[pb])

    def wait_wb(l, pb):
        pltpu.make_async_copy(oblk_v.at[pb], out_hbm.at[l, :, wid],
                              wb_sems[pb]).wait()

    def transpose_block(lp, pb):
        def bc_body(bc, carry):
            mvec = msk_s[lp, pl.ds(bc * LANES, LANES)] != 0
            row_vec = bc * LANES + iota
            for d in range(D):
                col_vec = jnp.full((LANES,), d, jnp.int32)
                v = plsc.load_gather(stag_v.at[pb], [row_vec, col_vec])
                v = jnp.where(mvec, v, 0.0)
                oblk_v[pb, d // 8, d % 8, pl.ds(bc * LANES, LANES)] = v
            return carry

        lax.fori_loop(0, BW // LANES, bc_body, 0)

    def slab_body(lt, carry):
        pltpu.sync_copy(sem_hbm.at[lt, wid], sem_s)
        pltpu.sync_copy(tt_hbm.at[lt, wid], tt_s)
        pltpu.sync_copy(msk_hbm.at[lt, wid], msk_s)
        compute_idx(lt, 0, 0)
        issue_gather(0)
        for lp in range(8):
            pb = lp & 1
            l = lt * 8 + lp
            if lp < 7:
                compute_idx(lt, lp + 1, pb ^ 1)
                issue_gather(pb ^ 1)
            wait_gather(pb)
            if lp >= 2:
                wait_wb(l - 2, pb)
            else:
                @pl.when(lt >= 1)
                def _():
                    wait_wb(l - 2, pb)
            transpose_block(lp, pb)
            issue_wb(l, pb)
        return carry

    lax.fori_loop(0, LT, slab_body, 0)

    wait_wb(L - 2, 0)
    wait_wb(L - 1, 1)


@jax.jit
def _run(sem4, tt4, msk4, emb):
    mesh = plsc.VectorSubcoreMesh(core_axis_name="c", subcore_axis_name="s")
    f = functools.partial(
        pl.kernel,
        mesh=mesh,
        out_type=jax.ShapeDtypeStruct((L, 8, NW, 8, BW), jnp.float32),
        scratch_types=[
            pltpu.VMEM((8, BW), jnp.int32),
            pltpu.VMEM((8, BW), jnp.int32),
            pltpu.VMEM((8, BW), jnp.int32),
            pltpu.VMEM((2, BW), jnp.int32),
            pltpu.VMEM((2, BW, D), jnp.float32),
            pltpu.VMEM((2, 8, 8, BW), jnp.float32),
            pltpu.SemaphoreType.DMA,
            pltpu.SemaphoreType.DMA,
            pltpu.SemaphoreType.DMA,
            pltpu.SemaphoreType.DMA,
        ],
        compiler_params=pltpu.CompilerParams(use_tc_tiling_on_sc=False,
                                             needs_layout_passes=False),
    )(_sc_gather)
    return f(sem4, tt4, msk4, emb)


def _as_slabs(x):
    return x.T.reshape(LT, 8, NW, BW).transpose(0, 2, 1, 3)


def kernel(sem_ids, token_type_ids, seq_mask, emb):
    sem4 = _as_slabs(sem_ids)
    tt4 = _as_slabs(token_type_ids)
    msk4 = _as_slabs(seq_mask.astype(jnp.int32))
    out5 = _run(sem4, tt4, msk4, emb)
    return out5.transpose(2, 4, 0, 1, 3).reshape(B, L, D)


# speedup vs baseline: 1.5840x; 1.5840x over previous
import functools

import jax
import jax.numpy as jnp
from jax import lax
from jax.experimental import pallas as pl
from jax.experimental.pallas import tpu as pltpu
from jax.experimental.pallas import tpu_sc as plsc

NUM_EMB = 100000
SEM_DIM = 8
D = 64
PAD_IDX = SEM_DIM * NUM_EMB
B, L = 4096, 200
N = B * L

NW = 32
BW = B // NW
LT = L // 8
LANES = 16
SPREAD_MASK = 524288 - 1


def _sc_gather(sem_hbm, tt_hbm, msk_hbm, table_hbm, out_hbm,
               sem_s, tt_s, msk_s, idx_v, stag_v, oblk_v,
               g_sem0, g_sem1, wb_sem0, wb_sem1):
    wid = lax.axis_index("s") * 2 + lax.axis_index("c")
    g_sems = (g_sem0, g_sem1)
    wb_sems = (wb_sem0, wb_sem1)
    iota = lax.iota(jnp.int32, LANES)

    def compute_idx(lt, lp, pb):
        l = lt * 8 + lp
        for g in range(BW // LANES):
            sl = pl.ds(g * LANES, LANES)
            ids = tt_s[lp, sl] * NUM_EMB + sem_s[lp, sl]
            pos = (l * B + wid * BW + g * LANES + iota) & SPREAD_MASK
            idx_v[pb, sl] = jnp.where(msk_s[lp, sl] != 0, ids, pos)

    def issue_gather(pb):
        pltpu.async_copy(table_hbm.at[idx_v.at[pb]],
                         stag_v.at[pb, pl.ds(0, BW)], g_sems[pb])

    def wait_gather(pb):
        pltpu.make_async_copy(table_hbm.at[idx_v.at[pb]],
                              stag_v.at[pb, pl.ds(0, BW)], g_sems[pb]).wait()

    def issue_wb(l, pb):
        pltpu.async_copy(oblk_v.at[pb], out_hbm.at[l, :, wid], wb_sems[pb])

    def wait_wb(l, pb):
        pltpu.make_async_copy(oblk_v.at[pb], out_hbm.at[l, :, wid],
                              wb_sems[pb]).wait()

    def transpose_block(lp, pb):
        def bc_body(bc, carry):
            mvec = msk_s[lp, pl.ds(bc * LANES, LANES)] != 0
            bvec = bc * LANES + iota
            row_m = jnp.where(mvec, bvec, BW)
            for d in range(D):
                colg = (iota + d) & (D - 1)
                v = plsc.load_gather(stag_v.at[pb], [row_m, colg])
                plsc.store_scatter(oblk_v.at[pb],
                                   [colg >> 3, colg & 7, bvec], v)
            return carry

        lax.fori_loop(0, BW // LANES, bc_body, 0)

    zeros16 = jnp.zeros((LANES,), jnp.float32)
    for pb in range(2):
        for i in range(D // LANES):
            stag_v[pb, BW, pl.ds(i * LANES, LANES)] = zeros16

    def slab_body(lt, carry):
        pltpu.sync_copy(sem_hbm.at[lt, wid], sem_s)
        pltpu.sync_copy(tt_hbm.at[lt, wid], tt_s)
        pltpu.sync_copy(msk_hbm.at[lt, wid], msk_s)
        compute_idx(lt, 0, 0)
        issue_gather(0)
        for lp in range(8):
            pb = lp & 1
            l = lt * 8 + lp
            if lp < 7:
                compute_idx(lt, lp + 1, pb ^ 1)
                issue_gather(pb ^ 1)
            wait_gather(pb)
            if lp >= 2:
                wait_wb(l - 2, pb)
            else:
                @pl.when(lt >= 1)
                def _():
                    wait_wb(l - 2, pb)
            transpose_block(lp, pb)
            issue_wb(l, pb)
        return carry

    lax.fori_loop(0, LT, slab_body, 0)

    wait_wb(L - 2, 0)
    wait_wb(L - 1, 1)


@jax.jit
def _run(sem4, tt4, msk4, emb):
    mesh = plsc.VectorSubcoreMesh(core_axis_name="c", subcore_axis_name="s")
    f = functools.partial(
        pl.kernel,
        mesh=mesh,
        out_type=jax.ShapeDtypeStruct((L, 8, NW, 8, BW), jnp.float32),
        scratch_types=[
            pltpu.VMEM((8, BW), jnp.int32),
            pltpu.VMEM((8, BW), jnp.int32),
            pltpu.VMEM((8, BW), jnp.int32),
            pltpu.VMEM((2, BW), jnp.int32),
            pltpu.VMEM((2, BW + 1, D), jnp.float32),
            pltpu.VMEM((2, 8, 8, BW), jnp.float32),
            pltpu.SemaphoreType.DMA,
            pltpu.SemaphoreType.DMA,
            pltpu.SemaphoreType.DMA,
            pltpu.SemaphoreType.DMA,
        ],
        compiler_params=pltpu.CompilerParams(use_tc_tiling_on_sc=False,
                                             needs_layout_passes=False),
    )(_sc_gather)
    return f(sem4, tt4, msk4, emb)


def _as_slabs(x):
    return x.T.reshape(LT, 8, NW, BW).transpose(0, 2, 1, 3)


def kernel(sem_ids, token_type_ids, seq_mask, emb):
    sem4 = _as_slabs(sem_ids)
    tt4 = _as_slabs(token_type_ids)
    msk4 = _as_slabs(seq_mask.astype(jnp.int32))
    out5 = _run(sem4, tt4, msk4, emb)
    return out5.transpose(2, 4, 0, 1, 3).reshape(B, L, D)
